# baseline (speedup 1.0000x reference)
import functools

import jax
import jax.numpy as jnp
from jax import lax
from jax.experimental import pallas as pl
from jax.experimental.pallas import tpu as pltpu
from jax.experimental.pallas import tpu_sc as plsc

_N, _FEAT, _HID, _NC, _TOPK, _CH, _OC = 50000, 2048, 256, 64, 10, 16, 2
_BN = 1000
_NBLK = _N // _BN
_NSUB = 16
_NW = 32
_NPAD = 50176
_CHW = _NPAD // _NW
_NVW = _CHW // 16
_HIGH = lax.Precision.HIGHEST


def _score_body(x_ref, w3_ref, o_ref):
    x = x_ref[...]
    wd = w3_ref[1, :] - w3_ref[0, :]
    acc = x[:, 0:128] * wd[0:128][None, :]
    for g in range(1, _FEAT // 128):
        lo = g * 128
        acc = acc + x[:, lo:lo + 128] * wd[lo:lo + 128][None, :]
    ones = jnp.ones((128, 1), jnp.float32)
    s = jnp.dot(acc, ones, precision=_HIGH,
                preferred_element_type=jnp.float32)
    o_ref[...] = s.reshape(1, 1, _BN)


def _scores(x_path, W3):
    return pl.pallas_call(
        _score_body,
        grid=(_NBLK,),
        in_specs=[
            pl.BlockSpec((_BN, _FEAT), lambda i: (i, 0)),
            pl.BlockSpec((2, _FEAT), lambda i: (0, 0)),
        ],
        out_specs=pl.BlockSpec((1, 1, _BN), lambda i: (i, 0, 0)),
        out_shape=jax.ShapeDtypeStruct((_NBLK, 1, _BN), jnp.float32),
        compiler_params=pltpu.CompilerParams(
            dimension_semantics=("arbitrary",)),
    )(x_path, W3)


def _sc_body(scores_hbm, vals_hbm, idx_hbm, chunk_v, bv_v, bi_v):
    cid = lax.axis_index("c")
    sid = lax.axis_index("s")
    wid = cid * _NSUB + sid
    base = wid * _CHW
    pltpu.sync_copy(scores_hbm.at[pl.ds(base, _CHW)], chunk_v)

    neg = jnp.float32(-jnp.inf)
    lane = lax.iota(jnp.int32, 16)

    init = (tuple(jnp.full((16,), neg, jnp.float32) for _ in range(_TOPK)),
            tuple(jnp.zeros((16,), jnp.int32) for _ in range(_TOPK)))

    def scan_step(i, carry):
        vals, idxs = carry
        v = chunk_v[pl.ds(i * 16, 16)]
        vi = lane + (base + i * 16)
        nv, ni = [], []
        for j in range(_TOPK):
            gt = v > vals[j]
            nv.append(jnp.where(gt, v, vals[j]))
            ni.append(jnp.where(gt, vi, idxs[j]))
            v = jnp.where(gt, vals[j], v)
            vi = jnp.where(gt, idxs[j], vi)
        return tuple(nv), tuple(ni)

    vals, idxs = lax.fori_loop(0, _NVW, scan_step, init)

    for j in range(_TOPK):
        bv_v[j, :] = vals[j]
        bi_v[j, :] = idxs[j]
    pltpu.sync_copy(bv_v, vals_hbm.at[wid])
    pltpu.sync_copy(bi_v, idx_hbm.at[wid])


def _sc_topk_candidates(scores_pad):
    mesh = plsc.VectorSubcoreMesh(core_axis_name="c", subcore_axis_name="s")
    f = functools.partial(
        pl.kernel,
        out_type=(
            jax.ShapeDtypeStruct((_NW, _TOPK, 16), jnp.float32),
            jax.ShapeDtypeStruct((_NW, _TOPK, 16), jnp.int32),
        ),
        mesh=mesh,
        compiler_params=pltpu.CompilerParams(needs_layout_passes=False),
        scratch_types=[
            pltpu.VMEM((_CHW,), jnp.float32),
            pltpu.VMEM((_TOPK, 16), jnp.float32),
            pltpu.VMEM((_TOPK, 16), jnp.int32),
        ],
    )(_sc_body)
    return f(scores_pad)


def _merge_body(v_ref, i_ref, o_ref):
    neg = jnp.float32(-jnp.inf)
    bigi = jnp.int32(2**31 - 1)
    vals = v_ref[...]
    idx = i_ref[...]
    for p in range(_TOPK):
        mx = jnp.max(vals)
        fid = jnp.min(jnp.where(vals == mx, idx, bigi))
        o_ref[p] = fid
        vals = jnp.where(idx == fid, neg, vals)
    for p in range(_TOPK, 16):
        o_ref[p] = jnp.int32(0)


def _merge_topk(cand_vals, cand_idx):
    return pl.pallas_call(
        _merge_body,
        out_shape=jax.ShapeDtypeStruct((16,), jnp.int32),
        out_specs=pl.BlockSpec(memory_space=pltpu.SMEM),
    )(cand_vals.reshape(_NW * _TOPK // 8, 128),
      cand_idx.reshape(_NW * _TOPK // 8, 128))


def _finish_body(cv_ref, ci_ref, x_ref, proto_ref, w2_ref, b2_ref, wr_ref,
                 br_ref, wc_ref, bc_ref, logits_ref, prob_ref, yhat_ref,
                 sim_ref, buf_ref, sems):
    neg = jnp.float32(-jnp.inf)
    bigi = jnp.int32(2**31 - 1)
    vals = cv_ref[...]
    idx = ci_ref[...]
    ids, cps = [], []
    for i in range(_TOPK):
        mx = jnp.max(vals)
        fid = jnp.min(jnp.where(vals == mx, idx, bigi))
        vals = jnp.where(idx == fid, neg, vals)
        grp = pl.multiple_of((fid // 8) * 8, 8)
        cp = pltpu.make_async_copy(
            x_ref.at[pl.ds(grp, 8), :], buf_ref.at[i], sems.at[i])
        cp.start()
        ids.append(fid)
        cps.append(cp)
    rows = []
    for i in range(_TOPK):
        cps[i].wait()
        rows.append(buf_ref[i, pl.ds(ids[i] % 8, 1), :])
    m = jnp.concatenate(rows, axis=0)

    dn = (((1,), (1,)), ((), ()))
    f = lax.dot_general(m, w2_ref[...], dn, precision=_HIGH,
                        preferred_element_type=jnp.float32) + b2_ref[...][None, :]
    p = lax.dot_general(proto_ref[...], w2_ref[...], dn, precision=_HIGH,
                        preferred_element_type=jnp.float32) + b2_ref[...][None, :]
    diff = f[:, None, :] - p[None, :, :] + 1e-6
    sim = jnp.sqrt(jnp.sum(diff * diff, axis=-1))
    sim = sim / jnp.max(sim, axis=1, keepdims=True)
    simc = jnp.mean(sim, axis=0, keepdims=True)
    sim_ref[...] = simc
    h = lax.dot_general(simc, wr_ref[...], dn, precision=_HIGH,
                        preferred_element_type=jnp.float32) + br_ref[...][None, :]
    h = jnp.maximum(h, 0.0)
    bl = lax.dot_general(h, wc_ref[...], dn, precision=_HIGH,
                         preferred_element_type=jnp.float32) + bc_ref[...][None, :]
    logits_ref[...] = bl
    e = jnp.exp(bl - jnp.max(bl, axis=1, keepdims=True))
    prob = e / jnp.sum(e, axis=1, keepdims=True)
    prob_ref[...] = prob
    yhat_ref[0] = jnp.where(prob[0, 1] > prob[0, 0], 1, 0).astype(jnp.int32)


def _finish(cand_vals, cand_idx, x_path, prototype, W2, b2, Wr, br, Wc, bc):
    return pl.pallas_call(
        _finish_body,
        in_specs=[
            pl.BlockSpec(memory_space=pltpu.VMEM),
            pl.BlockSpec(memory_space=pltpu.VMEM),
            pl.BlockSpec(memory_space=pl.ANY),
            pl.BlockSpec(memory_space=pltpu.VMEM),
            pl.BlockSpec(memory_space=pltpu.VMEM),
            pl.BlockSpec(memory_space=pltpu.VMEM),
            pl.BlockSpec(memory_space=pltpu.VMEM),
            pl.BlockSpec(memory_space=pltpu.VMEM),
            pl.BlockSpec(memory_space=pltpu.VMEM),
            pl.BlockSpec(memory_space=pltpu.VMEM),
        ],
        out_shape=(
            jax.ShapeDtypeStruct((1, _OC), jnp.float32),
            jax.ShapeDtypeStruct((1, _OC), jnp.float32),
            jax.ShapeDtypeStruct((1,), jnp.int32),
            jax.ShapeDtypeStruct((1, _NC), jnp.float32),
        ),
        out_specs=(
            pl.BlockSpec(memory_space=pltpu.VMEM),
            pl.BlockSpec(memory_space=pltpu.VMEM),
            pl.BlockSpec(memory_space=pltpu.SMEM),
            pl.BlockSpec(memory_space=pltpu.VMEM),
        ),
        scratch_shapes=[
            pltpu.VMEM((_TOPK, 8, _FEAT), jnp.float32),
            pltpu.SemaphoreType.DMA((_TOPK,)),
        ],
    )(cand_vals.reshape(_NW * _TOPK // 8, 128),
      cand_idx.reshape(_NW * _TOPK // 8, 128),
      x_path, prototype, W2, b2, Wr, br, Wc, bc)


def _tail_body(m_ref, proto_ref, w2_ref, b2_ref, wr_ref, br_ref,
               wc_ref, bc_ref, logits_ref, prob_ref, yhat_ref, sim_ref):
    dn = (((1,), (1,)), ((), ()))
    f = lax.dot_general(m_ref[...], w2_ref[...], dn, precision=_HIGH,
                        preferred_element_type=jnp.float32) + b2_ref[...][None, :]
    p = lax.dot_general(proto_ref[...], w2_ref[...], dn, precision=_HIGH,
                        preferred_element_type=jnp.float32) + b2_ref[...][None, :]
    diff = f[:, None, :] - p[None, :, :] + 1e-6
    sim = jnp.sqrt(jnp.sum(diff * diff, axis=-1))
    sim = sim / jnp.max(sim, axis=1, keepdims=True)
    simc = jnp.mean(sim, axis=0, keepdims=True)
    sim_ref[...] = simc
    h = lax.dot_general(simc, wr_ref[...], dn, precision=_HIGH,
                        preferred_element_type=jnp.float32) + br_ref[...][None, :]
    h = jnp.maximum(h, 0.0)
    bl = lax.dot_general(h, wc_ref[...], dn, precision=_HIGH,
                         preferred_element_type=jnp.float32) + bc_ref[...][None, :]
    logits_ref[...] = bl
    e = jnp.exp(bl - jnp.max(bl, axis=1, keepdims=True))
    prob = e / jnp.sum(e, axis=1, keepdims=True)
    prob_ref[...] = prob
    yhat_ref[0] = jnp.where(prob[0, 1] > prob[0, 0], 1, 0).astype(jnp.int32)


def _tail(mfeat, prototype, W2, b2, Wr, br, Wc, bc):
    return pl.pallas_call(
        _tail_body,
        out_shape=(
            jax.ShapeDtypeStruct((1, _OC), jnp.float32),
            jax.ShapeDtypeStruct((1, _OC), jnp.float32),
            jax.ShapeDtypeStruct((1,), jnp.int32),
            jax.ShapeDtypeStruct((1, _NC), jnp.float32),
        ),
        out_specs=(
            pl.BlockSpec(memory_space=pltpu.VMEM),
            pl.BlockSpec(memory_space=pltpu.VMEM),
            pl.BlockSpec(memory_space=pltpu.SMEM),
            pl.BlockSpec(memory_space=pltpu.VMEM),
        ),
    )(mfeat, prototype, W2, b2, Wr, br, Wc, bc)


def kernel(x_path, prototype, W3, b3, W2, b2, Wr, br, Wc, bc):
    del b3
    scores = _scores(x_path, W3).reshape(_N)
    scores_pad = jnp.pad(scores, (0, _NPAD - _N),
                         constant_values=-jnp.inf)
    cand_vals, cand_idx = _sc_topk_candidates(scores_pad)
    bag_logits, y_prob, y_hat, sim_coding = _finish(
        cand_vals, cand_idx, x_path, prototype, W2, b2, Wr, br, Wc, bc)
    return (bag_logits, y_prob, y_hat, sim_coding)


# speedup vs baseline: 1.0561x; 1.0561x over previous
import functools

import jax
import jax.numpy as jnp
from jax import lax
from jax.experimental import pallas as pl
from jax.experimental.pallas import tpu as pltpu
from jax.experimental.pallas import tpu_sc as plsc

_N, _FEAT, _HID, _NC, _TOPK, _CH, _OC = 50000, 2048, 256, 64, 10, 16, 2
_BN = 2000
_NBLK = _N // _BN
_NSUB = 16
_NW = 32
_NPAD = 50176
_CHW = _NPAD // _NW
_NVW = _CHW // 16
_HIGH = lax.Precision.HIGHEST


def _score_body(x_ref, w3_ref, o_ref):
    x = x_ref[...]
    wd = w3_ref[1, :] - w3_ref[0, :]
    acc = x[:, 0:128] * wd[0:128][None, :]
    for g in range(1, _FEAT // 128):
        lo = g * 128
        acc = acc + x[:, lo:lo + 128] * wd[lo:lo + 128][None, :]
    ones = jnp.ones((128, 1), jnp.float32)
    s = jnp.dot(acc, ones, precision=_HIGH,
                preferred_element_type=jnp.float32)
    o_ref[...] = s.reshape(1, 1, _BN)


def _scores(x_path, W3):
    return pl.pallas_call(
        _score_body,
        grid=(_NBLK,),
        in_specs=[
            pl.BlockSpec((_BN, _FEAT), lambda i: (i, 0)),
            pl.BlockSpec((2, _FEAT), lambda i: (0, 0)),
        ],
        out_specs=pl.BlockSpec((1, 1, _BN), lambda i: (i, 0, 0)),
        out_shape=jax.ShapeDtypeStruct((_NBLK, 1, _BN), jnp.float32),
        compiler_params=pltpu.CompilerParams(
            dimension_semantics=("arbitrary",)),
    )(x_path, W3)


def _sc_body(scores_hbm, vals_hbm, idx_hbm, chunk_v, bv_v, bi_v):
    cid = lax.axis_index("c")
    sid = lax.axis_index("s")
    wid = cid * _NSUB + sid
    base = wid * _CHW
    pltpu.sync_copy(scores_hbm.at[pl.ds(base, _CHW)], chunk_v)

    neg = jnp.float32(-jnp.inf)
    lane = lax.iota(jnp.int32, 16)

    init = (tuple(jnp.full((16,), neg, jnp.float32) for _ in range(_TOPK)),
            tuple(jnp.zeros((16,), jnp.int32) for _ in range(_TOPK)))

    def scan_step(i, carry):
        vals, idxs = carry
        v = chunk_v[pl.ds(i * 16, 16)]
        vi = lane + (base + i * 16)
        nv, ni = [], []
        for j in range(_TOPK):
            gt = v > vals[j]
            nv.append(jnp.where(gt, v, vals[j]))
            ni.append(jnp.where(gt, vi, idxs[j]))
            v = jnp.where(gt, vals[j], v)
            vi = jnp.where(gt, idxs[j], vi)
        return tuple(nv), tuple(ni)

    vals, idxs = lax.fori_loop(0, _NVW, scan_step, init)

    for j in range(_TOPK):
        bv_v[j, :] = vals[j]
        bi_v[j, :] = idxs[j]
    pltpu.sync_copy(bv_v, vals_hbm.at[wid])
    pltpu.sync_copy(bi_v, idx_hbm.at[wid])


def _sc_topk_candidates(scores_pad):
    mesh = plsc.VectorSubcoreMesh(core_axis_name="c", subcore_axis_name="s")
    f = functools.partial(
        pl.kernel,
        out_type=(
            jax.ShapeDtypeStruct((_NW, _TOPK, 16), jnp.float32),
            jax.ShapeDtypeStruct((_NW, _TOPK, 16), jnp.int32),
        ),
        mesh=mesh,
        compiler_params=pltpu.CompilerParams(needs_layout_passes=False),
        scratch_types=[
            pltpu.VMEM((_CHW,), jnp.float32),
            pltpu.VMEM((_TOPK, 16), jnp.float32),
            pltpu.VMEM((_TOPK, 16), jnp.int32),
        ],
    )(_sc_body)
    return f(scores_pad)


def _finish_body(cv_ref, ci_ref, x_ref, proto_ref, w2_ref, b2_ref, wr_ref,
                 br_ref, wc_ref, bc_ref, logits_ref, prob_ref, yhat_ref,
                 sim_ref, buf_ref, sems):
    neg = jnp.float32(-jnp.inf)
    bigi = jnp.int32(2**31 - 1)
    vals = cv_ref[...]
    idx = ci_ref[...]
    ids, cps = [], []
    for i in range(_TOPK):
        mx = jnp.max(vals)
        fid = jnp.min(jnp.where(vals == mx, idx, bigi))
        vals = jnp.where(idx == fid, neg, vals)
        grp = pl.multiple_of((fid // 8) * 8, 8)
        cp = pltpu.make_async_copy(
            x_ref.at[pl.ds(grp, 8), :], buf_ref.at[i], sems.at[i])
        cp.start()
        ids.append(fid)
        cps.append(cp)
    rows = []
    for i in range(_TOPK):
        cps[i].wait()
        rows.append(buf_ref[i, pl.ds(ids[i] % 8, 1), :])
    m = jnp.concatenate(rows, axis=0)

    dn = (((1,), (1,)), ((), ()))
    f = lax.dot_general(m, w2_ref[...], dn,
                        preferred_element_type=jnp.float32) + b2_ref[...][None, :]
    p = lax.dot_general(proto_ref[...], w2_ref[...], dn,
                        preferred_element_type=jnp.float32) + b2_ref[...][None, :]
    diff = f[:, None, :] - p[None, :, :] + 1e-6
    sim = jnp.sqrt(jnp.sum(diff * diff, axis=-1))
    sim = sim / jnp.max(sim, axis=1, keepdims=True)
    simc = jnp.mean(sim, axis=0, keepdims=True)
    sim_ref[...] = simc
    h = lax.dot_general(simc, wr_ref[...], dn,
                        preferred_element_type=jnp.float32) + br_ref[...][None, :]
    h = jnp.maximum(h, 0.0)
    bl = lax.dot_general(h, wc_ref[...], dn,
                         preferred_element_type=jnp.float32) + bc_ref[...][None, :]
    logits_ref[...] = bl
    e = jnp.exp(bl - jnp.max(bl, axis=1, keepdims=True))
    prob = e / jnp.sum(e, axis=1, keepdims=True)
    prob_ref[...] = prob
    yhat_ref[0] = jnp.where(prob[0, 1] > prob[0, 0], 1, 0).astype(jnp.int32)


def _finish(cand_vals, cand_idx, x_path, prototype, W2, b2, Wr, br, Wc, bc):
    return pl.pallas_call(
        _finish_body,
        in_specs=[
            pl.BlockSpec(memory_space=pltpu.VMEM),
            pl.BlockSpec(memory_space=pltpu.VMEM),
            pl.BlockSpec(memory_space=pl.ANY),
            pl.BlockSpec(memory_space=pltpu.VMEM),
            pl.BlockSpec(memory_space=pltpu.VMEM),
            pl.BlockSpec(memory_space=pltpu.VMEM),
            pl.BlockSpec(memory_space=pltpu.VMEM),
            pl.BlockSpec(memory_space=pltpu.VMEM),
            pl.BlockSpec(memory_space=pltpu.VMEM),
            pl.BlockSpec(memory_space=pltpu.VMEM),
        ],
        out_shape=(
            jax.ShapeDtypeStruct((1, _OC), jnp.float32),
            jax.ShapeDtypeStruct((1, _OC), jnp.float32),
            jax.ShapeDtypeStruct((1,), jnp.int32),
            jax.ShapeDtypeStruct((1, _NC), jnp.float32),
        ),
        out_specs=(
            pl.BlockSpec(memory_space=pltpu.VMEM),
            pl.BlockSpec(memory_space=pltpu.VMEM),
            pl.BlockSpec(memory_space=pltpu.SMEM),
            pl.BlockSpec(memory_space=pltpu.VMEM),
        ),
        scratch_shapes=[
            pltpu.VMEM((_TOPK, 8, _FEAT), jnp.float32),
            pltpu.SemaphoreType.DMA((_TOPK,)),
        ],
    )(cand_vals.reshape(_NW * _TOPK // 8, 128),
      cand_idx.reshape(_NW * _TOPK // 8, 128),
      x_path, prototype, W2, b2, Wr, br, Wc, bc)


def kernel(x_path, prototype, W3, b3, W2, b2, Wr, br, Wc, bc):
    del b3
    scores = _scores(x_path, W3).reshape(_N)
    scores_pad = jnp.pad(scores, (0, _NPAD - _N),
                         constant_values=-jnp.inf)
    cand_vals, cand_idx = _sc_topk_candidates(scores_pad)
    bag_logits, y_prob, y_hat, sim_coding = _finish(
        cand_vals, cand_idx, x_path, prototype, W2, b2, Wr, br, Wc, bc)
    return (bag_logits, y_prob, y_hat, sim_coding)
